# baseline (speedup 1.0000x reference)
import functools

import jax
import jax.numpy as jnp
from jax import lax
from jax.experimental import pallas as pl
from jax.experimental.pallas import tpu as pltpu
from jax.experimental.pallas import tpu_sc as plsc


RB = 256


def _topk_body(xr_ref, xa_ref, idx_ref):
    b = pl.program_id(0)
    rb = pl.program_id(1)
    xr = xr_ref[0]
    xa = xa_ref[0]
    n = xa.shape[0]

    x2r = jnp.sum(xr * xr, axis=1)
    x2a = jnp.sum(xa * xa, axis=1)
    gm2 = lax.dot_general(xr * -2.0, xa, (((1,), (1,)), ((), ())),
                          preferred_element_type=jnp.float32)
    d2 = (x2r[:, None] + x2a[None, :]) + gm2
    d2 = jnp.sqrt(jnp.maximum(d2, 0.0))

    col = lax.broadcasted_iota(jnp.int32, (RB, n), 1)
    row = rb * RB + lax.broadcasted_iota(jnp.int32, (RB, n), 0)
    colf = col.astype(jnp.float32)
    big = jnp.float32(3.0e38)
    nf = jnp.float32(n)
    d2 = jnp.where(col == row, big, d2)

    offs = b * n
    for t in range(3):
        m = jnp.min(d2, axis=1)
        aminf = jnp.min(jnp.where(d2 == m[:, None], colf, nf), axis=1)
        idx_ref[0, t] = (aminf.astype(jnp.int32) + offs)[:, None]
        if t < 2:
            d2 = jnp.where(colf == aminf[:, None], big, d2)


def _top3_indices(features):
    B, N, D = features.shape
    grid = (B, N // RB)
    return pl.pallas_call(
        _topk_body,
        grid=grid,
        in_specs=[
            pl.BlockSpec((1, RB, D), lambda b, r: (b, r, 0)),
            pl.BlockSpec((1, N, D), lambda b, r: (b, 0, 0)),
        ],
        out_specs=pl.BlockSpec((1, 3, RB, 1), lambda b, r: (b, 0, r, 0)),
        out_shape=jax.ShapeDtypeStruct((B, 3, N, 1), jnp.int32),
    )(features, features)


_G = 16


def _make_sc_mix(BN, N, D, rows_per_w, info):
    NC = info.num_cores
    n_chunks = rows_per_w // _G
    n_pairs = n_chunks // 2
    mesh = plsc.VectorSubcoreMesh(core_axis_name="c", subcore_axis_name="s")
    groups = _G * D // 16

    @functools.partial(
        pl.kernel,
        mesh=mesh,
        out_type=jax.ShapeDtypeStruct((BN, D), jnp.float32),
        scratch_types=[
            pltpu.VMEM((rows_per_w,), jnp.int32),
            pltpu.VMEM((rows_per_w,), jnp.int32),
            pltpu.VMEM((rows_per_w,), jnp.int32),
            pltpu.VMEM((16,), jnp.float32),
            pltpu.VMEM((_G, D), jnp.float32),
            pltpu.VMEM((_G, D), jnp.float32),
            pltpu.VMEM((_G, D), jnp.float32),
            pltpu.VMEM((_G, D), jnp.float32),
            pltpu.VMEM((_G, D), jnp.float32),
            pltpu.VMEM((_G, D), jnp.float32),
            pltpu.VMEM((_G, D), jnp.float32),
            pltpu.VMEM((_G, D), jnp.float32),
            pltpu.VMEM((_G, D), jnp.float32),
            pltpu.VMEM((_G, D), jnp.float32),
            pltpu.SemaphoreType.DMA,
            pltpu.SemaphoreType.DMA,
            pltpu.SemaphoreType.DMA,
            pltpu.SemaphoreType.DMA,
        ],
    )
    def sc_mix(feat_hbm, idx_hbm, sv_hbm, out_hbm,
               i0_v, i1_v, i2_v, sv_v,
               oA, g0A, g1A, g2A, obA,
               oB, g0B, g1B, g2B, obB,
               inA, inB, outA, outB):
        wid = lax.axis_index("s") * NC + lax.axis_index("c")
        base = wid * rows_per_w
        b = base // N
        n0 = base - b * N
        pltpu.sync_copy(idx_hbm.at[pl.ds((b * 3 + 0) * N + n0, rows_per_w)], i0_v)
        pltpu.sync_copy(idx_hbm.at[pl.ds((b * 3 + 1) * N + n0, rows_per_w)], i1_v)
        pltpu.sync_copy(idx_hbm.at[pl.ds((b * 3 + 2) * N + n0, rows_per_w)], i2_v)
        pltpu.sync_copy(sv_hbm, sv_v)
        s = jnp.clip(sv_v[...], 0.0, 1.0)
        ws = 1.0 - s
        wn = s * jnp.float32(1.0 / 3.0)

        def issue_in(c, o_b, g0_b, g1_b, g2_b, sem):
            rbase = base + c * _G
            pltpu.async_copy(feat_hbm.at[pl.ds(rbase, _G)], o_b, sem)
            pltpu.async_copy(feat_hbm.at[i0_v[pl.ds(c * _G, _G)]], g0_b, sem)
            pltpu.async_copy(feat_hbm.at[i1_v[pl.ds(c * _G, _G)]], g1_b, sem)
            pltpu.async_copy(feat_hbm.at[i2_v[pl.ds(c * _G, _G)]], g2_b, sem)

        def wait_in(o_b, g0_b, g1_b, g2_b, sem):
            for buf in (o_b, g0_b, g1_b, g2_b):
                pltpu.make_async_copy(feat_hbm.at[pl.ds(0, _G)], buf, sem).wait()

        def compute(o_b, g0_b, g1_b, g2_b, ob_b):
            @plsc.parallel_loop(0, groups, 1, unroll=8)
            def _(i):
                r = lax.shift_right_logical(i, 5)
                sl = pl.ds((i & 31) * 16, 16)
                acc = g0_b[r, sl] + g1_b[r, sl] + g2_b[r, sl]
                ob_b[r, sl] = ws * o_b[r, sl] + wn * acc

        def issue_out(c, ob_b, sem):
            pltpu.async_copy(ob_b, out_hbm.at[pl.ds(base + c * _G, _G)], sem)

        def wait_out(ob_b, sem):
            pltpu.make_async_copy(ob_b, out_hbm.at[pl.ds(0, _G)], sem).wait()

        issue_in(0, oA, g0A, g1A, g2A, inA)
        issue_in(1, oB, g0B, g1B, g2B, inB)

        def pair(g, carry):
            cA = 2 * g
            cB = cA + 1
            wait_in(oA, g0A, g1A, g2A, inA)
            pl.when(g > 0)(lambda: wait_out(obA, outA))
            compute(oA, g0A, g1A, g2A, obA)
            issue_out(cA, obA, outA)
            pl.when(g < n_pairs - 1)(
                lambda: issue_in(cA + 2, oA, g0A, g1A, g2A, inA))

            wait_in(oB, g0B, g1B, g2B, inB)
            pl.when(g > 0)(lambda: wait_out(obB, outB))
            compute(oB, g0B, g1B, g2B, obB)
            issue_out(cB, obB, outB)
            pl.when(g < n_pairs - 1)(
                lambda: issue_in(cB + 2, oB, g0B, g1B, g2B, inB))
            return carry

        lax.fori_loop(0, n_pairs, pair, 0)
        wait_out(obA, outA)
        wait_out(obB, outB)

    return sc_mix


_NSPLIT = 2


def kernel(features, strength):
    B, N, D = features.shape
    info = plsc.get_sparse_core_info()
    NW = info.num_cores * info.num_subcores

    sv = jnp.full((16,), strength, jnp.float32)

    nsplit = _NSPLIT if B % _NSPLIT == 0 else 1
    bs = B // nsplit
    BNs = bs * N
    rows_per_w = BNs // NW
    sc_mix = _make_sc_mix(BNs, N, D, rows_per_w, info)

    outs = []
    for i in range(nsplit):
        fpart = features[i * bs:(i + 1) * bs]
        idx = _top3_indices(fpart)
        out = sc_mix(fpart.reshape(BNs, D), idx.reshape(3 * BNs), sv)
        outs.append(out.reshape(bs, N, D))
    return jnp.concatenate(outs, axis=0) if nsplit > 1 else outs[0]


# speedup vs baseline: 1.1475x; 1.1475x over previous
import functools

import jax
import jax.numpy as jnp
from jax import lax
from jax.experimental import pallas as pl
from jax.experimental.pallas import tpu as pltpu
from jax.experimental.pallas import tpu_sc as plsc


RB = 512


def _topk_body(xr_ref, xa_ref, idx_ref):
    b = pl.program_id(0)
    rb = pl.program_id(1)
    xr = xr_ref[0]
    xa = xa_ref[0]
    n = xa.shape[0]

    x2r = jnp.sum(xr * xr, axis=1)
    x2a = jnp.sum(xa * xa, axis=1)
    gm2 = lax.dot_general(xr * -2.0, xa, (((1,), (1,)), ((), ())),
                          preferred_element_type=jnp.float32)
    d2 = (x2r[:, None] + x2a[None, :]) + gm2
    d2 = jnp.sqrt(jnp.maximum(d2, 0.0))

    col = lax.broadcasted_iota(jnp.int32, (RB, n), 1)
    row = rb * RB + lax.broadcasted_iota(jnp.int32, (RB, n), 0)
    colf = col.astype(jnp.float32)
    big = jnp.float32(3.0e38)
    nf = jnp.float32(n)
    d2 = jnp.where(col == row, big, d2)

    offs = b * n
    for t in range(3):
        m = jnp.min(d2, axis=1)
        aminf = jnp.min(jnp.where(d2 == m[:, None], colf, nf), axis=1)
        idx_ref[0, t] = (aminf.astype(jnp.int32) + offs)[:, None]
        if t < 2:
            d2 = jnp.where(colf == aminf[:, None], big, d2)


def _top3_indices(features):
    B, N, D = features.shape
    grid = (B, N // RB)
    return pl.pallas_call(
        _topk_body,
        grid=grid,
        in_specs=[
            pl.BlockSpec((1, RB, D), lambda b, r: (b, r, 0)),
            pl.BlockSpec((1, N, D), lambda b, r: (b, 0, 0)),
        ],
        out_specs=pl.BlockSpec((1, 3, RB, 1), lambda b, r: (b, 0, r, 0)),
        out_shape=jax.ShapeDtypeStruct((B, 3, N, 1), jnp.int32),
    )(features, features)


_G = 16


def _make_sc_mix(BN, N, D, rows_per_w, info):
    NC = info.num_cores
    n_chunks = rows_per_w // _G
    n_pairs = n_chunks // 2
    mesh = plsc.VectorSubcoreMesh(core_axis_name="c", subcore_axis_name="s")
    groups = _G * D // 16

    @functools.partial(
        pl.kernel,
        mesh=mesh,
        out_type=jax.ShapeDtypeStruct((BN, D), jnp.float32),
        scratch_types=[
            pltpu.VMEM((rows_per_w,), jnp.int32),
            pltpu.VMEM((rows_per_w,), jnp.int32),
            pltpu.VMEM((rows_per_w,), jnp.int32),
            pltpu.VMEM((16,), jnp.float32),
            pltpu.VMEM((_G, D), jnp.float32),
            pltpu.VMEM((_G, D), jnp.float32),
            pltpu.VMEM((_G, D), jnp.float32),
            pltpu.VMEM((_G, D), jnp.float32),
            pltpu.VMEM((_G, D), jnp.float32),
            pltpu.VMEM((_G, D), jnp.float32),
            pltpu.VMEM((_G, D), jnp.float32),
            pltpu.VMEM((_G, D), jnp.float32),
            pltpu.VMEM((_G, D), jnp.float32),
            pltpu.VMEM((_G, D), jnp.float32),
            pltpu.SemaphoreType.DMA,
            pltpu.SemaphoreType.DMA,
            pltpu.SemaphoreType.DMA,
            pltpu.SemaphoreType.DMA,
        ],
    )
    def sc_mix(feat_hbm, idx_hbm, sv_hbm, out_hbm,
               i0_v, i1_v, i2_v, sv_v,
               oA, g0A, g1A, g2A, obA,
               oB, g0B, g1B, g2B, obB,
               inA, inB, outA, outB):
        wid = lax.axis_index("s") * NC + lax.axis_index("c")
        base = wid * rows_per_w
        b = base // N
        n0 = base - b * N
        pltpu.sync_copy(idx_hbm.at[pl.ds((b * 3 + 0) * N + n0, rows_per_w)], i0_v)
        pltpu.sync_copy(idx_hbm.at[pl.ds((b * 3 + 1) * N + n0, rows_per_w)], i1_v)
        pltpu.sync_copy(idx_hbm.at[pl.ds((b * 3 + 2) * N + n0, rows_per_w)], i2_v)
        pltpu.sync_copy(sv_hbm, sv_v)
        s = jnp.clip(sv_v[...], 0.0, 1.0)
        ws = 1.0 - s
        wn = s * jnp.float32(1.0 / 3.0)

        def issue_in(c, o_b, g0_b, g1_b, g2_b, sem):
            rbase = base + c * _G
            pltpu.async_copy(feat_hbm.at[pl.ds(rbase, _G)], o_b, sem)
            pltpu.async_copy(feat_hbm.at[i0_v[pl.ds(c * _G, _G)]], g0_b, sem)
            pltpu.async_copy(feat_hbm.at[i1_v[pl.ds(c * _G, _G)]], g1_b, sem)
            pltpu.async_copy(feat_hbm.at[i2_v[pl.ds(c * _G, _G)]], g2_b, sem)

        def wait_in(o_b, g0_b, g1_b, g2_b, sem):
            for buf in (o_b, g0_b, g1_b, g2_b):
                pltpu.make_async_copy(feat_hbm.at[pl.ds(0, _G)], buf, sem).wait()

        def compute(o_b, g0_b, g1_b, g2_b, ob_b):
            @plsc.parallel_loop(0, groups, 1, unroll=8)
            def _(i):
                r = lax.shift_right_logical(i, 5)
                sl = pl.ds((i & 31) * 16, 16)
                acc = g0_b[r, sl] + g1_b[r, sl] + g2_b[r, sl]
                ob_b[r, sl] = ws * o_b[r, sl] + wn * acc

        def issue_out(c, ob_b, sem):
            pltpu.async_copy(ob_b, out_hbm.at[pl.ds(base + c * _G, _G)], sem)

        def wait_out(ob_b, sem):
            pltpu.make_async_copy(ob_b, out_hbm.at[pl.ds(0, _G)], sem).wait()

        issue_in(0, oA, g0A, g1A, g2A, inA)
        issue_in(1, oB, g0B, g1B, g2B, inB)

        def pair(g, carry):
            cA = 2 * g
            cB = cA + 1
            wait_in(oA, g0A, g1A, g2A, inA)
            pl.when(g > 0)(lambda: wait_out(obA, outA))
            compute(oA, g0A, g1A, g2A, obA)
            issue_out(cA, obA, outA)
            pl.when(g < n_pairs - 1)(
                lambda: issue_in(cA + 2, oA, g0A, g1A, g2A, inA))

            wait_in(oB, g0B, g1B, g2B, inB)
            pl.when(g > 0)(lambda: wait_out(obB, outB))
            compute(oB, g0B, g1B, g2B, obB)
            issue_out(cB, obB, outB)
            pl.when(g < n_pairs - 1)(
                lambda: issue_in(cB + 2, oB, g0B, g1B, g2B, inB))
            return carry

        lax.fori_loop(0, n_pairs, pair, 0)
        wait_out(obA, outA)
        wait_out(obB, outB)

    return sc_mix


_NSPLIT = 1


def kernel(features, strength):
    B, N, D = features.shape
    info = plsc.get_sparse_core_info()
    NW = info.num_cores * info.num_subcores

    sv = jnp.full((16,), strength, jnp.float32)

    nsplit = _NSPLIT if B % _NSPLIT == 0 else 1
    bs = B // nsplit
    BNs = bs * N
    rows_per_w = BNs // NW
    sc_mix = _make_sc_mix(BNs, N, D, rows_per_w, info)

    outs = []
    for i in range(nsplit):
        fpart = features[i * bs:(i + 1) * bs]
        idx = _top3_indices(fpart)
        out = sc_mix(fpart.reshape(BNs, D), idx.reshape(3 * BNs), sv)
        outs.append(out.reshape(bs, N, D))
    return jnp.concatenate(outs, axis=0) if nsplit > 1 else outs[0]


# speedup vs baseline: 1.1995x; 1.0453x over previous
import functools

import jax
import jax.numpy as jnp
from jax import lax
from jax.experimental import pallas as pl
from jax.experimental.pallas import tpu as pltpu
from jax.experimental.pallas import tpu_sc as plsc


RB = 1024


def _topk_body(xr_ref, xa_ref, idx_ref):
    b = pl.program_id(0)
    rb = pl.program_id(1)
    xr = xr_ref[0]
    xa = xa_ref[0]
    n = xa.shape[0]

    x2r = jnp.sum(xr * xr, axis=1)
    x2a = jnp.sum(xa * xa, axis=1)
    gm2 = lax.dot_general(xr * -2.0, xa, (((1,), (1,)), ((), ())),
                          preferred_element_type=jnp.float32)
    d2 = (x2r[:, None] + x2a[None, :]) + gm2
    d2 = jnp.sqrt(jnp.maximum(d2, 0.0))

    colf = lax.broadcasted_iota(jnp.int32, (RB, n), 1).astype(jnp.float32)
    rowf = (rb * RB + lax.broadcasted_iota(jnp.int32, (RB, 1), 0)).astype(jnp.float32)
    big = jnp.float32(3.0e38)
    nf = jnp.float32(n)
    d2 = jnp.where(colf == rowf, big, d2)

    offs = b * n
    for t in range(3):
        m = jnp.min(d2, axis=1)
        aminf = jnp.min(jnp.where(d2 == m[:, None], colf, nf), axis=1)
        idx_ref[0, t] = (aminf.astype(jnp.int32) + offs)[:, None]
        if t < 2:
            d2 = jnp.where(colf == aminf[:, None], big, d2)


def _top3_indices(features):
    B, N, D = features.shape
    grid = (B, N // RB)
    return pl.pallas_call(
        _topk_body,
        grid=grid,
        in_specs=[
            pl.BlockSpec((1, RB, D), lambda b, r: (b, r, 0)),
            pl.BlockSpec((1, N, D), lambda b, r: (b, 0, 0)),
        ],
        out_specs=pl.BlockSpec((1, 3, RB, 1), lambda b, r: (b, 0, r, 0)),
        out_shape=jax.ShapeDtypeStruct((B, 3, N, 1), jnp.int32),
    )(features, features)


_G = 16


def _make_sc_mix(BN, N, D, rows_per_w, info):
    NC = info.num_cores
    n_chunks = rows_per_w // _G
    n_pairs = n_chunks // 2
    mesh = plsc.VectorSubcoreMesh(core_axis_name="c", subcore_axis_name="s")
    groups = _G * D // 16

    @functools.partial(
        pl.kernel,
        mesh=mesh,
        out_type=jax.ShapeDtypeStruct((BN, D), jnp.float32),
        scratch_types=[
            pltpu.VMEM((rows_per_w,), jnp.int32),
            pltpu.VMEM((rows_per_w,), jnp.int32),
            pltpu.VMEM((rows_per_w,), jnp.int32),
            pltpu.VMEM((16,), jnp.float32),
            pltpu.VMEM((_G, D), jnp.float32),
            pltpu.VMEM((_G, D), jnp.float32),
            pltpu.VMEM((_G, D), jnp.float32),
            pltpu.VMEM((_G, D), jnp.float32),
            pltpu.VMEM((_G, D), jnp.float32),
            pltpu.VMEM((_G, D), jnp.float32),
            pltpu.VMEM((_G, D), jnp.float32),
            pltpu.VMEM((_G, D), jnp.float32),
            pltpu.VMEM((_G, D), jnp.float32),
            pltpu.VMEM((_G, D), jnp.float32),
            pltpu.SemaphoreType.DMA,
            pltpu.SemaphoreType.DMA,
            pltpu.SemaphoreType.DMA,
            pltpu.SemaphoreType.DMA,
        ],
    )
    def sc_mix(feat_hbm, idx_hbm, sv_hbm, out_hbm,
               i0_v, i1_v, i2_v, sv_v,
               oA, g0A, g1A, g2A, obA,
               oB, g0B, g1B, g2B, obB,
               inA, inB, outA, outB):
        wid = lax.axis_index("s") * NC + lax.axis_index("c")
        base = wid * rows_per_w
        b = base // N
        n0 = base - b * N
        pltpu.sync_copy(idx_hbm.at[pl.ds((b * 3 + 0) * N + n0, rows_per_w)], i0_v)
        pltpu.sync_copy(idx_hbm.at[pl.ds((b * 3 + 1) * N + n0, rows_per_w)], i1_v)
        pltpu.sync_copy(idx_hbm.at[pl.ds((b * 3 + 2) * N + n0, rows_per_w)], i2_v)
        pltpu.sync_copy(sv_hbm, sv_v)
        s = jnp.clip(sv_v[...], 0.0, 1.0)
        ws = 1.0 - s
        wn = s * jnp.float32(1.0 / 3.0)

        def issue_in(c, o_b, g0_b, g1_b, g2_b, sem):
            rbase = base + c * _G
            pltpu.async_copy(feat_hbm.at[pl.ds(rbase, _G)], o_b, sem)
            pltpu.async_copy(feat_hbm.at[i0_v[pl.ds(c * _G, _G)]], g0_b, sem)
            pltpu.async_copy(feat_hbm.at[i1_v[pl.ds(c * _G, _G)]], g1_b, sem)
            pltpu.async_copy(feat_hbm.at[i2_v[pl.ds(c * _G, _G)]], g2_b, sem)

        def wait_in(o_b, g0_b, g1_b, g2_b, sem):
            for buf in (o_b, g0_b, g1_b, g2_b):
                pltpu.make_async_copy(feat_hbm.at[pl.ds(0, _G)], buf, sem).wait()

        def compute(o_b, g0_b, g1_b, g2_b, ob_b):
            @plsc.parallel_loop(0, groups, 1, unroll=8)
            def _(i):
                r = lax.shift_right_logical(i, 5)
                sl = pl.ds((i & 31) * 16, 16)
                acc = g0_b[r, sl] + g1_b[r, sl] + g2_b[r, sl]
                ob_b[r, sl] = ws * o_b[r, sl] + wn * acc

        def issue_out(c, ob_b, sem):
            pltpu.async_copy(ob_b, out_hbm.at[pl.ds(base + c * _G, _G)], sem)

        def wait_out(ob_b, sem):
            pltpu.make_async_copy(ob_b, out_hbm.at[pl.ds(0, _G)], sem).wait()

        issue_in(0, oA, g0A, g1A, g2A, inA)
        issue_in(1, oB, g0B, g1B, g2B, inB)

        def pair(g, carry):
            cA = 2 * g
            cB = cA + 1
            wait_in(oA, g0A, g1A, g2A, inA)
            pl.when(g > 0)(lambda: wait_out(obA, outA))
            compute(oA, g0A, g1A, g2A, obA)
            issue_out(cA, obA, outA)
            pl.when(g < n_pairs - 1)(
                lambda: issue_in(cA + 2, oA, g0A, g1A, g2A, inA))

            wait_in(oB, g0B, g1B, g2B, inB)
            pl.when(g > 0)(lambda: wait_out(obB, outB))
            compute(oB, g0B, g1B, g2B, obB)
            issue_out(cB, obB, outB)
            pl.when(g < n_pairs - 1)(
                lambda: issue_in(cB + 2, oB, g0B, g1B, g2B, inB))
            return carry

        lax.fori_loop(0, n_pairs, pair, 0)
        wait_out(obA, outA)
        wait_out(obB, outB)

    return sc_mix


_NSPLIT = 1


def kernel(features, strength):
    B, N, D = features.shape
    info = plsc.get_sparse_core_info()
    NW = info.num_cores * info.num_subcores

    sv = jnp.full((16,), strength, jnp.float32)

    nsplit = _NSPLIT if B % _NSPLIT == 0 else 1
    bs = B // nsplit
    BNs = bs * N
    rows_per_w = BNs // NW
    sc_mix = _make_sc_mix(BNs, N, D, rows_per_w, info)

    outs = []
    for i in range(nsplit):
        fpart = features[i * bs:(i + 1) * bs]
        idx = _top3_indices(fpart)
        out = sc_mix(fpart.reshape(BNs, D), idx.reshape(3 * BNs), sv)
        outs.append(out.reshape(bs, N, D))
    return jnp.concatenate(outs, axis=0) if nsplit > 1 else outs[0]


# speedup vs baseline: 1.2095x; 1.0084x over previous
import functools

import jax
import jax.numpy as jnp
from jax import lax
from jax.experimental import pallas as pl
from jax.experimental.pallas import tpu as pltpu
from jax.experimental.pallas import tpu_sc as plsc


RB = 1024


def _topk_body(xr_ref, xa_ref, idx_ref):
    b = pl.program_id(0)
    rb = pl.program_id(1)
    xr = xr_ref[0]
    xa = xa_ref[0]
    n = xa.shape[0]

    x2r = jnp.sum(xr * xr, axis=1)
    x2a = jnp.sum(xa * xa, axis=1)
    gm2 = lax.dot_general(xr * -2.0, xa, (((1,), (1,)), ((), ())),
                          preferred_element_type=jnp.float32)
    col = lax.broadcasted_iota(jnp.int32, (RB, n), 1)
    row = rb * RB + lax.broadcasted_iota(jnp.int32, (RB, n), 0)
    colf = col.astype(jnp.float32)
    big = jnp.float32(3.0e38)
    nf = jnp.float32(n)
    d2 = jnp.where(
        col == row, big,
        jnp.sqrt(jnp.maximum((x2r[:, None] + x2a[None, :]) + gm2, 0.0)))

    offs = b * n
    for t in range(3):
        m = jnp.min(d2, axis=1)
        aminf = jnp.min(jnp.where(d2 == m[:, None], colf, nf), axis=1)
        idx_ref[0, t] = (aminf.astype(jnp.int32) + offs)[:, None]
        if t < 2:
            d2 = jnp.where(colf == aminf[:, None], big, d2)


def _top3_indices(features):
    B, N, D = features.shape
    grid = (B, N // RB)
    return pl.pallas_call(
        _topk_body,
        grid=grid,
        in_specs=[
            pl.BlockSpec((1, RB, D), lambda b, r: (b, r, 0)),
            pl.BlockSpec((1, N, D), lambda b, r: (b, 0, 0)),
        ],
        out_specs=pl.BlockSpec((1, 3, RB, 1), lambda b, r: (b, 0, r, 0)),
        out_shape=jax.ShapeDtypeStruct((B, 3, N, 1), jnp.int32),
    )(features, features)


_G = 16


def _make_sc_mix(BN, N, D, rows_per_w, info):
    NC = info.num_cores
    n_chunks = rows_per_w // _G
    n_pairs = n_chunks // 2
    mesh = plsc.VectorSubcoreMesh(core_axis_name="c", subcore_axis_name="s")
    groups = _G * D // 16

    @functools.partial(
        pl.kernel,
        mesh=mesh,
        out_type=jax.ShapeDtypeStruct((BN, D), jnp.float32),
        scratch_types=[
            pltpu.VMEM((rows_per_w,), jnp.int32),
            pltpu.VMEM((rows_per_w,), jnp.int32),
            pltpu.VMEM((rows_per_w,), jnp.int32),
            pltpu.VMEM((16,), jnp.float32),
            pltpu.VMEM((_G, D), jnp.float32),
            pltpu.VMEM((_G, D), jnp.float32),
            pltpu.VMEM((_G, D), jnp.float32),
            pltpu.VMEM((_G, D), jnp.float32),
            pltpu.VMEM((_G, D), jnp.float32),
            pltpu.VMEM((_G, D), jnp.float32),
            pltpu.VMEM((_G, D), jnp.float32),
            pltpu.VMEM((_G, D), jnp.float32),
            pltpu.VMEM((_G, D), jnp.float32),
            pltpu.VMEM((_G, D), jnp.float32),
            pltpu.SemaphoreType.DMA,
            pltpu.SemaphoreType.DMA,
            pltpu.SemaphoreType.DMA,
            pltpu.SemaphoreType.DMA,
        ],
    )
    def sc_mix(feat_hbm, idx_hbm, sv_hbm, out_hbm,
               i0_v, i1_v, i2_v, sv_v,
               oA, g0A, g1A, g2A, obA,
               oB, g0B, g1B, g2B, obB,
               inA, inB, outA, outB):
        wid = lax.axis_index("s") * NC + lax.axis_index("c")
        base = wid * rows_per_w
        b = base // N
        n0 = base - b * N
        pltpu.sync_copy(idx_hbm.at[pl.ds((b * 3 + 0) * N + n0, rows_per_w)], i0_v)
        pltpu.sync_copy(idx_hbm.at[pl.ds((b * 3 + 1) * N + n0, rows_per_w)], i1_v)
        pltpu.sync_copy(idx_hbm.at[pl.ds((b * 3 + 2) * N + n0, rows_per_w)], i2_v)
        pltpu.sync_copy(sv_hbm, sv_v)
        s = jnp.clip(sv_v[...], 0.0, 1.0)
        ws = 1.0 - s
        wn = s * jnp.float32(1.0 / 3.0)

        def issue_in(c, o_b, g0_b, g1_b, g2_b, sem):
            rbase = base + c * _G
            pltpu.async_copy(feat_hbm.at[pl.ds(rbase, _G)], o_b, sem)
            pltpu.async_copy(feat_hbm.at[i0_v[pl.ds(c * _G, _G)]], g0_b, sem)
            pltpu.async_copy(feat_hbm.at[i1_v[pl.ds(c * _G, _G)]], g1_b, sem)
            pltpu.async_copy(feat_hbm.at[i2_v[pl.ds(c * _G, _G)]], g2_b, sem)

        def wait_in(o_b, g0_b, g1_b, g2_b, sem):
            for buf in (o_b, g0_b, g1_b, g2_b):
                pltpu.make_async_copy(feat_hbm.at[pl.ds(0, _G)], buf, sem).wait()

        def compute(o_b, g0_b, g1_b, g2_b, ob_b):
            @plsc.parallel_loop(0, groups, 1, unroll=8)
            def _(i):
                r = lax.shift_right_logical(i, 5)
                sl = pl.ds((i & 31) * 16, 16)
                acc = g0_b[r, sl] + g1_b[r, sl] + g2_b[r, sl]
                ob_b[r, sl] = ws * o_b[r, sl] + wn * acc

        def issue_out(c, ob_b, sem):
            pltpu.async_copy(ob_b, out_hbm.at[pl.ds(base + c * _G, _G)], sem)

        def wait_out(ob_b, sem):
            pltpu.make_async_copy(ob_b, out_hbm.at[pl.ds(0, _G)], sem).wait()

        issue_in(0, oA, g0A, g1A, g2A, inA)
        issue_in(1, oB, g0B, g1B, g2B, inB)

        def pair(g, carry):
            cA = 2 * g
            cB = cA + 1
            wait_in(oA, g0A, g1A, g2A, inA)
            pl.when(g > 0)(lambda: wait_out(obA, outA))
            compute(oA, g0A, g1A, g2A, obA)
            issue_out(cA, obA, outA)
            pl.when(g < n_pairs - 1)(
                lambda: issue_in(cA + 2, oA, g0A, g1A, g2A, inA))

            wait_in(oB, g0B, g1B, g2B, inB)
            pl.when(g > 0)(lambda: wait_out(obB, outB))
            compute(oB, g0B, g1B, g2B, obB)
            issue_out(cB, obB, outB)
            pl.when(g < n_pairs - 1)(
                lambda: issue_in(cB + 2, oB, g0B, g1B, g2B, inB))
            return carry

        lax.fori_loop(0, n_pairs, pair, 0)
        wait_out(obA, outA)
        wait_out(obB, outB)

    return sc_mix


_NSPLIT = 1


def kernel(features, strength):
    B, N, D = features.shape
    info = plsc.get_sparse_core_info()
    NW = info.num_cores * info.num_subcores

    sv = jnp.full((16,), strength, jnp.float32)

    nsplit = _NSPLIT if B % _NSPLIT == 0 else 1
    bs = B // nsplit
    BNs = bs * N
    rows_per_w = BNs // NW
    sc_mix = _make_sc_mix(BNs, N, D, rows_per_w, info)

    outs = []
    for i in range(nsplit):
        fpart = features[i * bs:(i + 1) * bs]
        idx = _top3_indices(fpart)
        out = sc_mix(fpart.reshape(BNs, D), idx.reshape(3 * BNs), sv)
        outs.append(out.reshape(bs, N, D))
    return jnp.concatenate(outs, axis=0) if nsplit > 1 else outs[0]


# speedup vs baseline: 1.2191x; 1.0079x over previous
import functools

import jax
import jax.numpy as jnp
from jax import lax
from jax.experimental import pallas as pl
from jax.experimental.pallas import tpu as pltpu
from jax.experimental.pallas import tpu_sc as plsc


RB = 1024


def _topk_body(xr_ref, xa_ref, idx_ref):
    b = pl.program_id(0)
    rb = pl.program_id(1)
    xr = xr_ref[0]
    xa = xa_ref[0]
    n = xa.shape[0]

    x2r = jnp.sum(xr * xr, axis=1)
    x2a = jnp.sum(xa * xa, axis=1)
    gm2 = lax.dot_general(xr * -2.0, xa, (((1,), (1,)), ((), ())),
                          preferred_element_type=jnp.float32)
    col = lax.broadcasted_iota(jnp.int32, (RB, n), 1)
    row = rb * RB + lax.broadcasted_iota(jnp.int32, (RB, n), 0)
    colf = col.astype(jnp.float32)
    big = jnp.float32(3.0e38)
    nf = jnp.float32(n)
    d2 = jnp.where(
        col == row, big,
        jnp.sqrt(jnp.maximum((x2r[:, None] + x2a[None, :]) + gm2, 0.0)))

    offs = b * n
    for t in range(3):
        m = jnp.min(d2, axis=1)
        aminf = jnp.min(jnp.where(d2 == m[:, None], colf, nf), axis=1)
        idx_ref[0, t] = (aminf.astype(jnp.int32) + offs)[:, None]
        if t < 2:
            d2 = jnp.where(colf == aminf[:, None], big, d2)


def _top3_indices(features):
    B, N, D = features.shape
    grid = (B, N // RB)
    return pl.pallas_call(
        _topk_body,
        grid=grid,
        in_specs=[
            pl.BlockSpec((1, RB, D), lambda b, r: (b, r, 0)),
            pl.BlockSpec((1, N, D), lambda b, r: (b, 0, 0)),
        ],
        out_specs=pl.BlockSpec((1, 3, RB, 1), lambda b, r: (b, 0, r, 0)),
        out_shape=jax.ShapeDtypeStruct((B, 3, N, 1), jnp.int32),
    )(features, features)


_G = 16


def _make_sc_mix(BN, N, D, rows_per_w, info):
    NC = info.num_cores
    n_chunks = rows_per_w // _G
    n_pairs = n_chunks // 2
    mesh = plsc.VectorSubcoreMesh(core_axis_name="c", subcore_axis_name="s")
    groups = _G * D // 16

    @functools.partial(
        pl.kernel,
        mesh=mesh,
        out_type=jax.ShapeDtypeStruct((BN, D), jnp.float32),
        scratch_types=[
            pltpu.VMEM((rows_per_w,), jnp.int32),
            pltpu.VMEM((rows_per_w,), jnp.int32),
            pltpu.VMEM((rows_per_w,), jnp.int32),
            pltpu.VMEM((16,), jnp.float32),
            pltpu.VMEM((_G, D), jnp.float32),
            pltpu.VMEM((_G, D), jnp.float32),
            pltpu.VMEM((_G, D), jnp.float32),
            pltpu.VMEM((_G, D), jnp.float32),
            pltpu.VMEM((_G, D), jnp.float32),
            pltpu.VMEM((_G, D), jnp.float32),
            pltpu.VMEM((_G, D), jnp.float32),
            pltpu.VMEM((_G, D), jnp.float32),
            pltpu.VMEM((_G, D), jnp.float32),
            pltpu.VMEM((_G, D), jnp.float32),
            pltpu.SemaphoreType.DMA,
            pltpu.SemaphoreType.DMA,
            pltpu.SemaphoreType.DMA,
            pltpu.SemaphoreType.DMA,
        ],
    )
    def sc_mix(feat_hbm, idx_hbm, sv_hbm, out_hbm,
               i0_v, i1_v, i2_v, sv_v,
               oA, g0A, g1A, g2A, obA,
               oB, g0B, g1B, g2B, obB,
               inA, inB, outA, outB):
        wid = lax.axis_index("s") * NC + lax.axis_index("c")
        base = wid * rows_per_w
        b = base // N
        n0 = base - b * N
        cps = [
            pltpu.async_copy(
                idx_hbm.at[pl.ds((b * 3 + j) * N + n0, rows_per_w)], iv, inA)
            for j, iv in enumerate((i0_v, i1_v, i2_v))
        ]
        cps.append(pltpu.async_copy(sv_hbm, sv_v, inA))
        for cp in cps:
            cp.wait()
        s = jnp.clip(sv_v[...], 0.0, 1.0)
        ws = 1.0 - s
        wn = s * jnp.float32(1.0 / 3.0)

        def issue_in(c, o_b, g0_b, g1_b, g2_b, sem):
            rbase = base + c * _G
            pltpu.async_copy(feat_hbm.at[pl.ds(rbase, _G)], o_b, sem)
            pltpu.async_copy(feat_hbm.at[i0_v[pl.ds(c * _G, _G)]], g0_b, sem)
            pltpu.async_copy(feat_hbm.at[i1_v[pl.ds(c * _G, _G)]], g1_b, sem)
            pltpu.async_copy(feat_hbm.at[i2_v[pl.ds(c * _G, _G)]], g2_b, sem)

        def wait_in(o_b, g0_b, g1_b, g2_b, sem):
            for buf in (o_b, g0_b, g1_b, g2_b):
                pltpu.make_async_copy(feat_hbm.at[pl.ds(0, _G)], buf, sem).wait()

        def compute(o_b, g0_b, g1_b, g2_b, ob_b):
            @plsc.parallel_loop(0, groups, 1, unroll=16)
            def _(i):
                r = lax.shift_right_logical(i, 5)
                sl = pl.ds((i & 31) * 16, 16)
                acc = g0_b[r, sl] + g1_b[r, sl] + g2_b[r, sl]
                ob_b[r, sl] = ws * o_b[r, sl] + wn * acc

        def issue_out(c, ob_b, sem):
            pltpu.async_copy(ob_b, out_hbm.at[pl.ds(base + c * _G, _G)], sem)

        def wait_out(ob_b, sem):
            pltpu.make_async_copy(ob_b, out_hbm.at[pl.ds(0, _G)], sem).wait()

        issue_in(0, oA, g0A, g1A, g2A, inA)
        issue_in(1, oB, g0B, g1B, g2B, inB)

        def pair(g, carry):
            cA = 2 * g
            cB = cA + 1
            wait_in(oA, g0A, g1A, g2A, inA)
            pl.when(g > 0)(lambda: wait_out(obA, outA))
            compute(oA, g0A, g1A, g2A, obA)
            issue_out(cA, obA, outA)
            pl.when(g < n_pairs - 1)(
                lambda: issue_in(cA + 2, oA, g0A, g1A, g2A, inA))

            wait_in(oB, g0B, g1B, g2B, inB)
            pl.when(g > 0)(lambda: wait_out(obB, outB))
            compute(oB, g0B, g1B, g2B, obB)
            issue_out(cB, obB, outB)
            pl.when(g < n_pairs - 1)(
                lambda: issue_in(cB + 2, oB, g0B, g1B, g2B, inB))
            return carry

        lax.fori_loop(0, n_pairs, pair, 0)
        wait_out(obA, outA)
        wait_out(obB, outB)

    return sc_mix


_NSPLIT = 1


def kernel(features, strength):
    B, N, D = features.shape
    info = plsc.get_sparse_core_info()
    NW = info.num_cores * info.num_subcores

    sv = jnp.full((16,), strength, jnp.float32)

    nsplit = _NSPLIT if B % _NSPLIT == 0 else 1
    bs = B // nsplit
    BNs = bs * N
    rows_per_w = BNs // NW
    sc_mix = _make_sc_mix(BNs, N, D, rows_per_w, info)

    outs = []
    for i in range(nsplit):
        fpart = features[i * bs:(i + 1) * bs]
        idx = _top3_indices(fpart)
        out = sc_mix(fpart.reshape(BNs, D), idx.reshape(3 * BNs), sv)
        outs.append(out.reshape(bs, N, D))
    return jnp.concatenate(outs, axis=0) if nsplit > 1 else outs[0]
